# baseline (speedup 1.0000x reference)
import functools

import jax
import jax.numpy as jnp
from jax import lax
from jax.experimental import pallas as pl
from jax.experimental.pallas import tpu as pltpu
from jax.experimental.pallas import tpu_sc as plsc

NUM_NODES = 1000000
EMBED_DIM = 64
BATCH = 16384
N_FIELDS = 26

_CHUNK = 128
_TOTAL = BATCH * N_FIELDS
_NCHUNKS = _TOTAL // _CHUNK
_NW = 32
_CPW = _NCHUNKS // _NW


def _make_kernel():
    mesh = plsc.VectorSubcoreMesh(core_axis_name="c", subcore_axis_name="s")

    @functools.partial(
        pl.kernel,
        mesh=mesh,
        compiler_params=pltpu.CompilerParams(use_tc_tiling_on_sc=False),
        out_type=jax.ShapeDtypeStruct((_NCHUNKS, _CHUNK, EMBED_DIM), jnp.float32),
        scratch_types=[
            pltpu.VMEM((_CPW, _CHUNK), jnp.int32),
            pltpu.VMEM((_CHUNK, EMBED_DIM), jnp.float32),
            pltpu.SemaphoreType.DMA,
        ],
    )
    def gather_kernel(ids_hbm, table_hbm, out_hbm, idx_v, rows_v, sem):
        wid = lax.axis_index("s") * 2 + lax.axis_index("c")
        base = wid * _CPW
        pltpu.sync_copy(ids_hbm.at[pl.ds(base, _CPW)], idx_v)

        def body(j, carry):
            pltpu.async_copy(table_hbm.at[idx_v.at[j]], rows_v, sem).wait()
            pltpu.sync_copy(rows_v, out_hbm.at[base + j])
            return carry

        lax.fori_loop(0, _CPW, body, 0)

    return gather_kernel


_gather = _make_kernel()


def kernel(ids, weight):
    ids_flat = ids.astype(jnp.int32).reshape(_NCHUNKS, _CHUNK)
    out = _gather(ids_flat, weight)
    return out.reshape(BATCH, N_FIELDS, EMBED_DIM)


# speedup vs baseline: 1.0708x; 1.0708x over previous
import functools

import jax
import jax.numpy as jnp
from jax import lax
from jax.experimental import pallas as pl
from jax.experimental.pallas import tpu as pltpu
from jax.experimental.pallas import tpu_sc as plsc

NUM_NODES = 1000000
EMBED_DIM = 64
BATCH = 16384
N_FIELDS = 26

_CHUNK = 128
_TOTAL = BATCH * N_FIELDS
_NCHUNKS = _TOTAL // _CHUNK
_NW = 32
_CPW = _NCHUNKS // _NW
_GRP = 4
_NGRP = _CPW // _GRP
_NPAIR = _NGRP // 2


def _make_kernel():
    mesh = plsc.VectorSubcoreMesh(core_axis_name="c", subcore_axis_name="s")

    @functools.partial(
        pl.kernel,
        mesh=mesh,
        compiler_params=pltpu.CompilerParams(use_tc_tiling_on_sc=False),
        out_type=jax.ShapeDtypeStruct((_NCHUNKS, _CHUNK, EMBED_DIM), jnp.float32),
        scratch_types=[
            pltpu.VMEM((_CPW, _CHUNK), jnp.int32),
            pltpu.VMEM((2 * _GRP, _CHUNK, EMBED_DIM), jnp.float32),
            pltpu.SemaphoreType.DMA,
            pltpu.SemaphoreType.DMA,
            pltpu.SemaphoreType.DMA,
            pltpu.SemaphoreType.DMA,
        ],
    )
    def gather_kernel(ids_hbm, table_hbm, out_hbm, idx_v, rows_v,
                      sem_ga, sem_gb, sem_oa, sem_ob):
        wid = lax.axis_index("s") * 2 + lax.axis_index("c")
        base = wid * _CPW
        pltpu.sync_copy(ids_hbm.at[pl.ds(base, _CPW)], idx_v)

        def start_gathers(g, half, sem):
            for b in range(_GRP):
                pltpu.async_copy(
                    table_hbm.at[idx_v.at[g * _GRP + b]],
                    rows_v.at[half * _GRP + b], sem)

        def wait_gathers(half, sem):
            for b in range(_GRP):
                pltpu.make_async_copy(
                    table_hbm.at[idx_v.at[0]],
                    rows_v.at[half * _GRP + b], sem).wait()

        def start_outs(g, half, sem):
            for b in range(_GRP):
                j = g * _GRP + b
                pltpu.async_copy(
                    rows_v.at[half * _GRP + b], out_hbm.at[base + j], sem)

        def wait_outs(g, half, sem):
            for b in range(_GRP):
                j = g * _GRP + b
                pltpu.make_async_copy(
                    rows_v.at[half * _GRP + b], out_hbm.at[base + j],
                    sem).wait()

        start_gathers(0, 0, sem_ga)

        def body(k, carry):
            g0 = 2 * k
            g1 = 2 * k + 1
            wait_gathers(0, sem_ga)
            start_outs(g0, 0, sem_oa)
            @pl.when(k > 0)
            def _():
                wait_outs(g1 - 2, 1, sem_ob)
            start_gathers(g1, 1, sem_gb)
            wait_gathers(1, sem_gb)
            start_outs(g1, 1, sem_ob)
            @pl.when(k < _NPAIR - 1)
            def _():
                wait_outs(g0, 0, sem_oa)
                start_gathers(g0 + 2, 0, sem_ga)
            return carry

        lax.fori_loop(0, _NPAIR, body, 0)

        wait_outs(_NGRP - 2, 0, sem_oa)
        wait_outs(_NGRP - 1, 1, sem_ob)

    return gather_kernel


_gather = _make_kernel()


def kernel(ids, weight):
    ids_flat = ids.astype(jnp.int32).reshape(_NCHUNKS, _CHUNK)
    out = _gather(ids_flat, weight)
    return out.reshape(BATCH, N_FIELDS, EMBED_DIM)
